# Initial kernel scaffold; baseline (speedup 1.0000x reference)
import jax
import jax.numpy as jnp
from jax.experimental import pallas as pl


def kernel(indices, position_emb):
    raise NotImplementedError("write your pallas kernel here")


# speedup vs baseline: 6.1729x; 6.1729x over previous
import functools

import jax
import jax.numpy as jnp
from jax import lax
from jax.experimental import pallas as pl
from jax.experimental.pallas import tpu as pltpu
from jax.experimental.pallas import tpu_sc as plsc


def _make_sc_gather(B, D, chunk):
    info = plsc.get_sparse_core_info()
    nc, ns = info.num_cores, info.num_subcores
    nw = nc * ns
    assert B % nw == 0
    b_per_w = B // nw
    assert b_per_w % chunk == 0
    n_chunks = b_per_w // chunk
    mesh = plsc.VectorSubcoreMesh(core_axis_name="c", subcore_axis_name="s")

    @functools.partial(
        pl.kernel,
        mesh=mesh,
        compiler_params=pltpu.CompilerParams(use_tc_tiling_on_sc=False),
        out_type=jax.ShapeDtypeStruct((B, D), jnp.float32),
        scratch_types=[
            pltpu.VMEM((chunk,), jnp.int32),
            pltpu.VMEM((chunk, D), jnp.float32),
            pltpu.SemaphoreType.DMA,
        ],
    )
    def gather_kernel(idx_hbm, table_hbm, out_hbm, idx_v, rows_v, sem):
        wid = lax.axis_index("s") * nc + lax.axis_index("c")
        base = wid * b_per_w

        def body(i, carry):
            off = base + i * chunk
            pltpu.sync_copy(idx_hbm.at[pl.ds(off, chunk)], idx_v)
            pltpu.async_copy(table_hbm.at[idx_v], rows_v, sem).wait()
            pltpu.sync_copy(rows_v, out_hbm.at[pl.ds(off, chunk)])
            return carry

        lax.fori_loop(0, n_chunks, body, 0)

    return gather_kernel


def kernel(indices, position_emb):
    bsz, hist = indices.shape
    _, d = position_emb.shape
    flat_idx = indices.reshape(bsz * hist)
    fn = _make_sc_gather(bsz * hist, d, 3200)
    out = fn(flat_idx, position_emb)
    return out.reshape(bsz, hist, d)


# speedup vs baseline: 6.2173x; 1.0072x over previous
import functools

import jax
import jax.numpy as jnp
from jax import lax
from jax.experimental import pallas as pl
from jax.experimental.pallas import tpu as pltpu
from jax.experimental.pallas import tpu_sc as plsc


def _make_sc_gather(B, D, chunk):
    info = plsc.get_sparse_core_info()
    nc, ns = info.num_cores, info.num_subcores
    nw = nc * ns
    assert B % nw == 0
    b_per_w = B // nw
    assert b_per_w % chunk == 0
    n_chunks = b_per_w // chunk
    mesh = plsc.VectorSubcoreMesh(core_axis_name="c", subcore_axis_name="s")

    @functools.partial(
        pl.kernel,
        mesh=mesh,
        compiler_params=pltpu.CompilerParams(use_tc_tiling_on_sc=False),
        out_type=jax.ShapeDtypeStruct((B, D), jnp.float32),
        scratch_types=[
            pltpu.VMEM((chunk,), jnp.int32),
            pltpu.VMEM((chunk,), jnp.int32),
            pltpu.VMEM((chunk, D), jnp.float32),
            pltpu.VMEM((chunk, D), jnp.float32),
            pltpu.SemaphoreType.DMA,
            pltpu.SemaphoreType.DMA,
            pltpu.SemaphoreType.DMA,
            pltpu.SemaphoreType.DMA,
        ],
    )
    def gather_kernel(idx_hbm, table_hbm, out_hbm,
                      idx0, idx1, rows0, rows1, sg0, sg1, so0, so1):
        wid = lax.axis_index("s") * nc + lax.axis_index("c")
        base = wid * b_per_w
        idx_v = (idx0, idx1)
        rows_v = (rows0, rows1)
        sg = (sg0, sg1)
        so = (so0, so1)
        gathers = [None, None]
        stores = [None, None]
        for i in range(n_chunks):
            b = i & 1
            if stores[b] is not None:
                stores[b].wait()
            off = base + i * chunk
            pltpu.sync_copy(idx_hbm.at[pl.ds(off, chunk)], idx_v[b])
            gathers[b] = pltpu.async_copy(
                table_hbm.at[idx_v[b]], rows_v[b], sg[b])
            if i >= 1:
                pb = 1 - b
                gathers[pb].wait()
                stores[pb] = pltpu.async_copy(
                    rows_v[pb],
                    out_hbm.at[pl.ds(base + (i - 1) * chunk, chunk)],
                    so[pb])
        lb = (n_chunks - 1) & 1
        gathers[lb].wait()
        stores[lb] = pltpu.async_copy(
            rows_v[lb],
            out_hbm.at[pl.ds(base + (n_chunks - 1) * chunk, chunk)],
            so[lb])
        if stores[1 - lb] is not None:
            stores[1 - lb].wait()
        stores[lb].wait()

    return gather_kernel


def kernel(indices, position_emb):
    bsz, hist = indices.shape
    _, d = position_emb.shape
    flat_idx = indices.reshape(bsz * hist)
    fn = _make_sc_gather(bsz * hist, d, 1600)
    out = fn(flat_idx, position_emb)
    return out.reshape(bsz, hist, d)


# speedup vs baseline: 6.6730x; 1.0733x over previous
import functools

import jax
import jax.numpy as jnp
from jax import lax
from jax.experimental import pallas as pl
from jax.experimental.pallas import tpu as pltpu
from jax.experimental.pallas import tpu_sc as plsc


def _make_sc_gather(B, V, D, chunk):
    info = plsc.get_sparse_core_info()
    nc, ns = info.num_cores, info.num_subcores
    nw = nc * ns
    assert B % nw == 0
    b_per_w = B // nw
    assert b_per_w % chunk == 0
    n_chunks = b_per_w // chunk
    mesh = plsc.VectorSubcoreMesh(core_axis_name="c", subcore_axis_name="s")

    @functools.partial(
        pl.kernel,
        mesh=mesh,
        compiler_params=pltpu.CompilerParams(use_tc_tiling_on_sc=False),
        out_type=jax.ShapeDtypeStruct((B, D), jnp.float32),
        scratch_types=[
            pltpu.VMEM_SHARED((V, D), jnp.float32),
            pltpu.VMEM((chunk,), jnp.int32),
            pltpu.VMEM((chunk,), jnp.int32),
            pltpu.VMEM((chunk, D), jnp.float32),
            pltpu.VMEM((chunk, D), jnp.float32),
            pltpu.SemaphoreType.DMA,
            pltpu.SemaphoreType.DMA,
            pltpu.SemaphoreType.DMA,
            pltpu.SemaphoreType.DMA,
        ],
    )
    def gather_kernel(idx_hbm, table_hbm, out_hbm,
                      table_sh, idx0, idx1, rows0, rows1, sg0, sg1, so0, so1):
        sid = lax.axis_index("s")
        wid = sid * nc + lax.axis_index("c")
        base = wid * b_per_w

        @pl.when(sid == 0)
        def _():
            pltpu.sync_copy(table_hbm, table_sh)

        plsc.subcore_barrier()

        idx_v = (idx0, idx1)
        rows_v = (rows0, rows1)
        sg = (sg0, sg1)
        so = (so0, so1)
        gathers = [None, None]
        stores = [None, None]
        for i in range(n_chunks):
            b = i & 1
            if stores[b] is not None:
                stores[b].wait()
            off = base + i * chunk
            pltpu.sync_copy(idx_hbm.at[pl.ds(off, chunk)], idx_v[b])
            gathers[b] = pltpu.async_copy(
                table_sh.at[idx_v[b]], rows_v[b], sg[b])
            if i >= 1:
                pb = 1 - b
                gathers[pb].wait()
                stores[pb] = pltpu.async_copy(
                    rows_v[pb],
                    out_hbm.at[pl.ds(base + (i - 1) * chunk, chunk)],
                    so[pb])
        lb = (n_chunks - 1) & 1
        gathers[lb].wait()
        stores[lb] = pltpu.async_copy(
            rows_v[lb],
            out_hbm.at[pl.ds(base + (n_chunks - 1) * chunk, chunk)],
            so[lb])
        if stores[1 - lb] is not None:
            stores[1 - lb].wait()
        stores[lb].wait()

    return gather_kernel


def kernel(indices, position_emb):
    bsz, hist = indices.shape
    v, d = position_emb.shape
    flat_idx = indices.reshape(bsz * hist)
    fn = _make_sc_gather(bsz * hist, v, d, 1600)
    out = fn(flat_idx, position_emb)
    return out.reshape(bsz, hist, d)
